# Initial kernel scaffold; baseline (speedup 1.0000x reference)
import jax
import jax.numpy as jnp
from jax.experimental import pallas as pl


def kernel(x, targets, table):
    raise NotImplementedError("write your pallas kernel here")


# speedup vs baseline: 18.4978x; 18.4978x over previous
import functools

import jax
import jax.numpy as jnp
from jax import lax
from jax.experimental import pallas as pl
from jax.experimental.pallas import tpu as pltpu
from jax.experimental.pallas import tpu_sc as plsc

VOCAB = 1000
NTOK = 1024 * 50
LANES = 16


def _adj_body(tbl_ref, out_ref):
    t = tbl_ref[...]
    m = jnp.max(t, axis=1)
    s = jnp.sum(jnp.exp(t - m[:, None]), axis=1)
    lz = jnp.log(s) + m
    out_ref[...] = lz[:, None] - t


def _final_body(p_ref, out_ref):
    out_ref[0, 0] = jnp.sum(p_ref[...]) * (1.0 / NTOK)


def _make_token_kernel(nc, ns):
    nw = nc * ns
    tpw = NTOK // nw
    nvec = tpw // LANES
    full, rem = divmod(tpw, 128)
    mesh = plsc.VectorSubcoreMesh(core_axis_name="c", subcore_axis_name="s")

    @functools.partial(
        pl.kernel,
        mesh=mesh,
        out_type=jax.ShapeDtypeStruct((nc * LANES,), jnp.float32),
        scratch_types=[
            pltpu.VMEM((tpw,), jnp.int32),
            pltpu.VMEM((tpw,), jnp.int32),
            pltpu.VMEM((tpw,), jnp.int32),
            pltpu.VMEM((tpw,), jnp.float32),
            pltpu.VMEM((LANES,), jnp.float32),
            pltpu.VMEM((ns * LANES,), jnp.float32),
            pltpu.VMEM_SHARED((ns * LANES,), jnp.float32),
            pltpu.SemaphoreType.DMA,
        ],
    )
    def token_kernel(x_hbm, t_hbm, adj_hbm, out_hbm,
                     xv, tv, idxv, pickv, accv, allv, shared, sem):
        cid = lax.axis_index("c")
        sid = lax.axis_index("s")
        wid = cid * ns + sid
        base = wid * tpw

        pltpu.sync_copy(x_hbm.at[pl.ds(base, tpw)], xv)
        pltpu.sync_copy(t_hbm.at[pl.ds(base, tpw)], tv)

        def idx_body(i, carry):
            off = i * LANES
            xc = xv[pl.ds(off, LANES)]
            tc = tv[pl.ds(off, LANES)]
            idxv[pl.ds(off, LANES)] = xc * VOCAB + tc
            return carry

        lax.fori_loop(0, nvec, idx_body, 0)

        handles = []
        for j in range(full):
            handles.append(pltpu.async_copy(
                adj_hbm.at[idxv.at[pl.ds(j * 128, 128)]],
                pickv.at[pl.ds(j * 128, 128)], sem))
        if rem:
            handles.append(pltpu.async_copy(
                adj_hbm.at[idxv.at[pl.ds(full * 128, rem)]],
                pickv.at[pl.ds(full * 128, rem)], sem))
        for h in handles:
            h.wait()

        def acc_body(i, acc):
            off = i * LANES
            return acc + pickv[pl.ds(off, LANES)]

        acc = lax.fori_loop(0, nvec, acc_body,
                            jnp.zeros((LANES,), jnp.float32))
        accv[...] = acc

        pltpu.sync_copy(accv, shared.at[pl.ds(sid * LANES, LANES)])
        plsc.subcore_barrier()

        @pl.when(sid == 0)
        def _():
            pltpu.sync_copy(shared, allv)

            def fold(i, tot):
                return tot + allv[pl.ds(i * LANES, LANES)]

            tot = lax.fori_loop(0, ns, fold,
                                jnp.zeros((LANES,), jnp.float32))
            accv[...] = tot
            pltpu.sync_copy(accv, out_hbm.at[pl.ds(cid * LANES, LANES)])

    return token_kernel


def kernel(x, targets, table):
    info = plsc.get_sparse_core_info()
    nc, ns = info.num_cores, info.num_subcores

    xf = x.reshape(-1).astype(jnp.int32)
    tf = targets.reshape(-1).astype(jnp.int32)

    adj = pl.pallas_call(
        _adj_body,
        out_shape=jax.ShapeDtypeStruct((VOCAB, VOCAB), jnp.float32),
    )(table)

    partials = _make_token_kernel(nc, ns)(xf, tf, adj.reshape(-1))

    loss = pl.pallas_call(
        _final_body,
        out_shape=jax.ShapeDtypeStruct((1, 1), jnp.float32),
        out_specs=pl.BlockSpec(memory_space=pltpu.SMEM),
    )(partials)
    return loss.reshape(())
